# Initial kernel scaffold; baseline (speedup 1.0000x reference)
import jax
import jax.numpy as jnp
from jax.experimental import pallas as pl


def kernel(x, edge_index, edge_pairs, W1l, W1r, b1, W2l, W2r, b2, Wp1, bp1, Wp2, bp2):
    raise NotImplementedError("write your pallas kernel here")


# speedup vs baseline: 7.5606x; 7.5606x over previous
import jax
import jax.numpy as jnp
from jax import lax
from jax.experimental import pallas as pl
from jax.experimental.pallas import tpu as pltpu
from jax.experimental.pallas import tpu_sc as plsc

NC = 2
NS = 16
NW = NC * NS

N_SUM = 10016
N_CNT = 10240
SINK = 10008
ROWS_PAD = 2512
BIG = 25


def _sc_mesh():
    return plsc.VectorSubcoreMesh(core_axis_name="c", subcore_axis_name="s")


def _segment_sum_sc(x, src2d, dst2d, zeros2d, zeros1d, with_cnt):
    n, d = x.shape
    rpt = 624
    tail = n - NS * rpt
    zpt = N_SUM - NS * rpt
    cpt = N_CNT // NS

    out_type = [jax.ShapeDtypeStruct((NC, n, d), jnp.float32)]
    scratch = [
        pltpu.VMEM((80, 128), jnp.int32),
        pltpu.VMEM((80, 128), jnp.int32),
        pltpu.VMEM((128, d), jnp.float32),
        pltpu.VMEM_SHARED((N_SUM, d), jnp.float32),
        pltpu.SemaphoreType.DMA,
        pltpu.SemaphoreType.DMA,
    ]
    if with_cnt:
        out_type.append(jax.ShapeDtypeStruct((NC, N_CNT), jnp.float32))
        scratch += [
            pltpu.VMEM((128,), jnp.float32),
            pltpu.VMEM_SHARED((N_CNT,), jnp.float32),
        ]

    def body(x_h, src_h, dst_h, z2_h, z1_h, *rest):
        if with_cnt:
            (sum_o, cnt_o, src_v, dst_v, buf_v, acc_s, sem0, sem1,
             ones_v, cacc_s) = rest
        else:
            sum_o, src_v, dst_v, buf_v, acc_s, sem0, sem1 = rest
        c = lax.axis_index("c")
        s = lax.axis_index("s")
        wid = c * NS + s

        pltpu.sync_copy(z2_h.at[pl.ds(s * rpt, rpt)],
                        acc_s.at[pl.ds(s * rpt, rpt)])

        @pl.when(s == 0)
        def _():
            pltpu.sync_copy(z2_h.at[pl.ds(NS * rpt, zpt)],
                            acc_s.at[pl.ds(NS * rpt, zpt)])

        if with_cnt:
            def fill_ones(i, carry):
                ones_v[pl.ds(i * 16, 16)] = jnp.full((16,), 1.0, jnp.float32)
                return carry
            lax.fori_loop(0, 8, fill_ones, 0)
            pltpu.sync_copy(z1_h.at[pl.ds(s * cpt, cpt)],
                            cacc_s.at[pl.ds(s * cpt, cpt)])
        plsc.subcore_barrier()

        nr = jnp.where(wid < BIG, 80, 72)
        base = jnp.minimum(wid, BIG) * 80 + jnp.maximum(wid - BIG, 0) * 72
        pltpu.sync_copy(src_h.at[pl.ds(base, 80)], src_v)
        pltpu.sync_copy(dst_h.at[pl.ds(base, 80)], dst_v)

        def step(j, carry):
            pltpu.async_copy(x_h.at[src_v.at[j]], buf_v, sem0).wait()
            pltpu.sync_copy(buf_v, acc_s.at[dst_v.at[j]], add=True)
            if with_cnt:
                pltpu.sync_copy(ones_v, cacc_s.at[dst_v.at[j]], add=True)
            return carry
        lax.fori_loop(0, nr, step, 0)

        plsc.subcore_barrier()

        pltpu.sync_copy(acc_s.at[pl.ds(s * rpt, rpt)],
                        sum_o.at[c, pl.ds(s * rpt, rpt)])

        @pl.when(s == 0)
        def _():
            pltpu.sync_copy(acc_s.at[pl.ds(NS * rpt, tail)],
                            sum_o.at[c, pl.ds(NS * rpt, tail)])

        if with_cnt:
            pltpu.sync_copy(cacc_s.at[pl.ds(s * cpt, cpt)],
                            cnt_o.at[c, pl.ds(s * cpt, cpt)])

    k = pl.kernel(body, out_type=tuple(out_type), mesh=_sc_mesh(),
                  scratch_types=tuple(scratch))
    return k(x, src2d, dst2d, zeros2d, zeros1d)


def _pair_gather_sc(h, pairs2d):
    n, d = h.shape
    rows = pairs2d.shape[1]
    rpw = rows // NW
    p = rows * 128

    def body(h_h, p_h, hs_o, hd_o, idxs_v, idxd_v, buf_v, sem0, sem1):
        c = lax.axis_index("c")
        s = lax.axis_index("s")
        wid = c * NS + s
        pltpu.sync_copy(p_h.at[0, pl.ds(wid * rpw, rpw)], idxs_v)
        pltpu.sync_copy(p_h.at[1, pl.ds(wid * rpw, rpw)], idxd_v)

        def step(j, carry):
            row = wid * rpw + j
            pltpu.async_copy(h_h.at[idxs_v.at[j]], buf_v.at[0], sem0).wait()
            pltpu.sync_copy(buf_v.at[0], hs_o.at[pl.ds(row * 128, 128)])
            pltpu.async_copy(h_h.at[idxd_v.at[j]], buf_v.at[1], sem1).wait()
            pltpu.sync_copy(buf_v.at[1], hd_o.at[pl.ds(row * 128, 128)])
            return carry
        lax.fori_loop(0, rpw, step, 0)

    k = pl.kernel(
        body,
        out_type=(jax.ShapeDtypeStruct((p, d), jnp.float32),
                  jax.ShapeDtypeStruct((p, d), jnp.float32)),
        mesh=_sc_mesh(),
        scratch_types=(
            pltpu.VMEM((rpw, 128), jnp.int32),
            pltpu.VMEM((rpw, 128), jnp.int32),
            pltpu.VMEM((2, 128, d), jnp.float32),
            pltpu.SemaphoreType.DMA,
            pltpu.SemaphoreType.DMA,
        ),
    )
    return k(h, pairs2d)


def _sage_dense_tc(sum2, cnt_t, x, wl, wr, b, relu):
    n, d = x.shape
    r = 1000

    def body(sum_ref, cnt_ref, x_ref, wl_ref, wr_ref, b_ref, o_ref):
        ssum = sum_ref[0] + sum_ref[1]
        cnt = cnt_ref[:, 0] + cnt_ref[:, 1]
        agg = ssum / jnp.clip(cnt, 1.0)[:, None]
        h = (jnp.dot(agg, wl_ref[...], preferred_element_type=jnp.float32)
             + jnp.dot(x_ref[...], wr_ref[...],
                       preferred_element_type=jnp.float32)
             + b_ref[...])
        if relu:
            h = jnp.maximum(h, 0.0)
        o_ref[...] = h

    return pl.pallas_call(
        body,
        grid=(n // r,),
        in_specs=[
            pl.BlockSpec((NC, r, d), lambda i: (0, i, 0)),
            pl.BlockSpec((r, NC), lambda i: (i, 0)),
            pl.BlockSpec((r, d), lambda i: (i, 0)),
            pl.BlockSpec((d, d), lambda i: (0, 0)),
            pl.BlockSpec((d, d), lambda i: (0, 0)),
            pl.BlockSpec((1, d), lambda i: (0, 0)),
        ],
        out_specs=pl.BlockSpec((r, d), lambda i: (i, 0)),
        out_shape=jax.ShapeDtypeStruct((n, d), jnp.float32),
    )(sum2, cnt_t, x, wl, wr, b)


def _predictor_tc(hs, hd, wpa, wpb, bp1, wp2, bp2):
    p, d = hs.shape
    q = 2048

    def body(hs_ref, hd_ref, wpa_ref, wpb_ref, bp1_ref, wp2_ref, bp2_ref,
             o_ref):
        z = (jnp.dot(hs_ref[...], wpa_ref[...],
                     preferred_element_type=jnp.float32)
             + jnp.dot(hd_ref[...], wpb_ref[...],
                       preferred_element_type=jnp.float32)
             + bp1_ref[...])
        z = jnp.maximum(z, 0.0)
        o = jnp.dot(z, wp2_ref[...], preferred_element_type=jnp.float32)
        o_ref[...] = jax.nn.sigmoid(o + bp2_ref[...])

    return pl.pallas_call(
        body,
        grid=(p // q,),
        in_specs=[
            pl.BlockSpec((q, d), lambda i: (i, 0)),
            pl.BlockSpec((q, d), lambda i: (i, 0)),
            pl.BlockSpec((d, d), lambda i: (0, 0)),
            pl.BlockSpec((d, d), lambda i: (0, 0)),
            pl.BlockSpec((1, d), lambda i: (0, 0)),
            pl.BlockSpec((d, 1), lambda i: (0, 0)),
            pl.BlockSpec((1, 1), lambda i: (0, 0)),
        ],
        out_specs=pl.BlockSpec((q, 1), lambda i: (i, 0)),
        out_shape=jax.ShapeDtypeStruct((p, 1), jnp.float32),
    )(hs, hd, wpa, wpb, bp1, wp2, bp2)


def kernel(x, edge_index, edge_pairs, W1l, W1r, b1, W2l, W2r, b2,
           Wp1, bp1, Wp2, bp2):
    n, d = x.shape
    rows = edge_index.shape[1] // 128
    src2d = jnp.pad(edge_index[0].reshape(rows, 128),
                    ((0, ROWS_PAD - rows), (0, 0)))
    dst2d = jnp.pad(edge_index[1].reshape(rows, 128),
                    ((0, ROWS_PAD - rows), (0, 0)), constant_values=SINK)
    zeros2d = jnp.zeros((N_SUM, d), jnp.float32)
    zeros1d = jnp.zeros((N_CNT,), jnp.float32)

    sum1, cnt = _segment_sum_sc(x, src2d, dst2d, zeros2d, zeros1d,
                                with_cnt=True)
    cnt_t = cnt[:, :n].T
    h1 = _sage_dense_tc(sum1, cnt_t, x, W1l, W1r, b1.reshape(1, -1),
                        relu=True)
    (sum2,) = _segment_sum_sc(h1, src2d, dst2d, zeros2d, zeros1d,
                              with_cnt=False)
    h = _sage_dense_tc(sum2, cnt_t, h1, W2l, W2r, b2.reshape(1, -1),
                       relu=False)

    pairs2d = edge_pairs.reshape(2, -1, 128)
    hs, hd = _pair_gather_sc(h, pairs2d)
    out = _predictor_tc(hs, hd, Wp1[:d], Wp1[d:], bp1.reshape(1, -1),
                        Wp2, bp2.reshape(1, 1))
    return out[:, 0]
